# baseline (speedup 1.0000x reference)
import functools

import jax
import jax.numpy as jnp
from jax import lax
from jax.experimental import pallas as pl
from jax.experimental.pallas import tpu as pltpu
from jax.experimental.pallas import tpu_sc as plsc

B, L = 16384, 200
N = B * L
V, D = 800, 128
NC, NS = 2, 16
NW = NC * NS
LANES = 16
CH = 2048
EPR = B // CH
TRIPS = N // (NW * CH)

ROWS = 64
ROW_TILES = 13


def _sc_body(xt_hbm, table_hbm, w_hbm, b_hbm, out_hbm,
             tab_v, w_v, b_v, q_local, q_shared, q_v,
             idx0, idx1, out0, out1, si0, si1, so0, so1):
    cid = lax.axis_index("c")
    sid = lax.axis_index("s")
    w = cid * NS + sid
    base = w * TRIPS

    def in_src(t):
        q = base + t
        r = lax.shift_right_logical(q, 3)
        e = lax.bitwise_and(q, EPR - 1)
        return xt_hbm.at[r, pl.ds(e * CH, CH)]

    def out_dst(t):
        return out_hbm.at[pl.ds((base + t) * CH, CH)]

    pltpu.async_copy(in_src(0), idx0, si0)
    pltpu.async_copy(in_src(1), idx1, si1)

    pltpu.sync_copy(w_hbm, w_v)
    pltpu.sync_copy(b_hbm, b_v)

    @pl.when(sid < ROW_TILES)
    def _compute_rows():
        row0 = jnp.minimum(sid * ROWS, V - ROWS)
        pltpu.sync_copy(table_hbm.at[pl.ds(row0, ROWS)], tab_v)
        bias = b_v[pl.ds(0, LANES)][0]
        lane = lax.broadcasted_iota(jnp.int32, (LANES,), 0)
        wd = [w_v[pl.ds((d // LANES) * LANES, LANES)][d % LANES] for d in range(D)]
        for g in range(ROWS // LANES):
            rows = lane + g * LANES
            acc = jnp.zeros((LANES,), jnp.float32)
            for d in range(D):
                col = plsc.load_gather(tab_v, [rows, jnp.full((LANES,), d, jnp.int32)])
                acc = acc + col * wd[d]
            q_local[pl.ds(g * LANES, LANES)] = jnp.maximum(acc + bias, 0.0)
        pltpu.sync_copy(q_local, q_shared.at[pl.ds(row0, ROWS)])

    plsc.subcore_barrier()
    pltpu.sync_copy(q_shared, q_v)

    def gather_chunk(idx_b, out_b):
        def _inner(m, carry):
            mb = m * (16 * LANES)
            for jb in range(16):
                iv = idx_b[pl.ds(mb + jb * LANES, LANES)]
                out_b[pl.ds(mb + jb * LANES, LANES)] = plsc.load_gather(q_v, [iv])
            return carry

        lax.fori_loop(0, CH // (16 * LANES), _inner, 0)

    def wait_in(idx_b, si_b):
        pltpu.make_async_copy(xt_hbm.at[0, pl.ds(0, CH)], idx_b, si_b).wait()

    def wait_out(out_b, so_b):
        pltpu.make_async_copy(out_b, out_hbm.at[pl.ds(0, CH)], so_b).wait()

    wait_in(idx0, si0)
    gather_chunk(idx0, out0)
    pltpu.async_copy(out0, out_dst(0), so0)
    pltpu.async_copy(in_src(2), idx0, si0)

    wait_in(idx1, si1)
    gather_chunk(idx1, out1)
    pltpu.async_copy(out1, out_dst(1), so1)
    pltpu.async_copy(in_src(3), idx1, si1)

    @pl.loop(2, TRIPS - 2, step=2)
    def _steady(t):
        wait_in(idx0, si0)
        wait_out(out0, so0)
        gather_chunk(idx0, out0)
        pltpu.async_copy(out0, out_dst(t), so0)
        pltpu.async_copy(in_src(t + 2), idx0, si0)

        wait_in(idx1, si1)
        wait_out(out1, so1)
        gather_chunk(idx1, out1)
        pltpu.async_copy(out1, out_dst(t + 1), so1)
        pltpu.async_copy(in_src(t + 3), idx1, si1)

    wait_in(idx0, si0)
    wait_out(out0, so0)
    gather_chunk(idx0, out0)
    pltpu.async_copy(out0, out_dst(TRIPS - 2), so0)

    wait_in(idx1, si1)
    wait_out(out1, so1)
    gather_chunk(idx1, out1)
    pltpu.async_copy(out1, out_dst(TRIPS - 1), so1)

    wait_out(out0, so0)
    wait_out(out1, so1)


_sc_gather = functools.partial(
    pl.kernel,
    out_type=jax.ShapeDtypeStruct((N,), jnp.float32),
    mesh=plsc.VectorSubcoreMesh(core_axis_name="c", subcore_axis_name="s"),
    compiler_params=pltpu.CompilerParams(needs_layout_passes=False),
    scratch_types=[
        pltpu.VMEM((ROWS, D), jnp.float32),
        pltpu.VMEM((D,), jnp.float32),
        pltpu.VMEM((LANES,), jnp.float32),
        pltpu.VMEM((ROWS,), jnp.float32),
        pltpu.VMEM_SHARED((V,), jnp.float32),
        pltpu.VMEM((V,), jnp.float32),
        pltpu.VMEM((CH,), jnp.int32),
        pltpu.VMEM((CH,), jnp.int32),
        pltpu.VMEM((CH,), jnp.float32),
        pltpu.VMEM((CH,), jnp.float32),
        pltpu.SemaphoreType.DMA,
        pltpu.SemaphoreType.DMA,
        pltpu.SemaphoreType.DMA,
        pltpu.SemaphoreType.DMA,
    ],
)(_sc_body)


def kernel(x1, table, W, b):
    xt = jnp.swapaxes(x1.astype(jnp.int32), 0, 1)
    w_vec = W.reshape(-1)
    b_vec = jnp.broadcast_to(b.reshape(-1), (LANES,))
    out_flat = _sc_gather(xt, table, w_vec, b_vec)
    return jnp.transpose(out_flat.reshape(L, B, 1), (1, 0, 2))


# speedup vs baseline: 1.0914x; 1.0914x over previous
import functools

import jax
import jax.numpy as jnp
from jax import lax
from jax.experimental import pallas as pl
from jax.experimental.pallas import tpu as pltpu
from jax.experimental.pallas import tpu_sc as plsc

B, L = 16384, 200
N = B * L
V, D = 800, 128
NC, NS = 2, 16
NW = NC * NS
LANES = 16
CH = 4096
EPR = B // CH
EPR_LOG = EPR.bit_length() - 1
TRIPS = N // (NW * CH)

ROWS = 64
ROW_TILES = 13


def _sc_body(xt_hbm, table_hbm, w_hbm, b_hbm, out_hbm,
             tab_v, w_v, b_v, q_local, q_shared, q_v,
             idx0, idx1, out0, out1, si0, si1, so0, so1):
    cid = lax.axis_index("c")
    sid = lax.axis_index("s")
    w = cid * NS + sid
    base = w * TRIPS

    def in_src(t):
        q = base + t
        r = lax.shift_right_logical(q, EPR_LOG)
        e = lax.bitwise_and(q, EPR - 1)
        return xt_hbm.at[r, pl.ds(e * CH, CH)]

    def out_dst(t):
        return out_hbm.at[pl.ds((base + t) * CH, CH)]

    pltpu.async_copy(in_src(0), idx0, si0)
    pltpu.async_copy(in_src(1), idx1, si1)

    pltpu.sync_copy(w_hbm, w_v)
    pltpu.sync_copy(b_hbm, b_v)

    @pl.when(sid < ROW_TILES)
    def _compute_rows():
        row0 = jnp.minimum(sid * ROWS, V - ROWS)
        pltpu.sync_copy(table_hbm.at[pl.ds(row0, ROWS)], tab_v)
        bias = b_v[pl.ds(0, LANES)][0]
        lane = lax.broadcasted_iota(jnp.int32, (LANES,), 0)
        wd = [w_v[pl.ds((d // LANES) * LANES, LANES)][d % LANES] for d in range(D)]
        for g in range(ROWS // LANES):
            rows = lane + g * LANES
            acc = jnp.zeros((LANES,), jnp.float32)
            for d in range(D):
                col = plsc.load_gather(tab_v, [rows, jnp.full((LANES,), d, jnp.int32)])
                acc = acc + col * wd[d]
            q_local[pl.ds(g * LANES, LANES)] = jnp.maximum(acc + bias, 0.0)
        pltpu.sync_copy(q_local, q_shared.at[pl.ds(row0, ROWS)])

    plsc.subcore_barrier()
    pltpu.sync_copy(q_shared, q_v)

    def gather_chunk(idx_b, out_b):
        def _inner(m, carry):
            mb = m * (16 * LANES)
            for jb in range(16):
                iv = idx_b[pl.ds(mb + jb * LANES, LANES)]
                out_b[pl.ds(mb + jb * LANES, LANES)] = plsc.load_gather(q_v, [iv])
            return carry

        lax.fori_loop(0, CH // (16 * LANES), _inner, 0)

    def wait_in(idx_b, si_b):
        pltpu.make_async_copy(xt_hbm.at[0, pl.ds(0, CH)], idx_b, si_b).wait()

    def wait_out(out_b, so_b):
        pltpu.make_async_copy(out_b, out_hbm.at[pl.ds(0, CH)], so_b).wait()

    wait_in(idx0, si0)
    gather_chunk(idx0, out0)
    pltpu.async_copy(out0, out_dst(0), so0)
    pltpu.async_copy(in_src(2), idx0, si0)

    wait_in(idx1, si1)
    gather_chunk(idx1, out1)
    pltpu.async_copy(out1, out_dst(1), so1)
    pltpu.async_copy(in_src(3), idx1, si1)

    @pl.loop(2, TRIPS - 3, step=2)
    def _steady(t):
        wait_in(idx0, si0)
        wait_out(out0, so0)
        gather_chunk(idx0, out0)
        pltpu.async_copy(out0, out_dst(t), so0)
        pltpu.async_copy(in_src(t + 2), idx0, si0)

        wait_in(idx1, si1)
        wait_out(out1, so1)
        gather_chunk(idx1, out1)
        pltpu.async_copy(out1, out_dst(t + 1), so1)
        pltpu.async_copy(in_src(t + 3), idx1, si1)

    wait_in(idx0, si0)
    wait_out(out0, so0)
    gather_chunk(idx0, out0)
    pltpu.async_copy(out0, out_dst(TRIPS - 3), so0)
    pltpu.async_copy(in_src(TRIPS - 1), idx0, si0)

    wait_in(idx1, si1)
    wait_out(out1, so1)
    gather_chunk(idx1, out1)
    pltpu.async_copy(out1, out_dst(TRIPS - 2), so1)

    wait_in(idx0, si0)
    wait_out(out0, so0)
    gather_chunk(idx0, out0)
    pltpu.async_copy(out0, out_dst(TRIPS - 1), so0)

    wait_out(out0, so0)
    wait_out(out1, so1)


_sc_gather = functools.partial(
    pl.kernel,
    out_type=jax.ShapeDtypeStruct((N,), jnp.float32),
    mesh=plsc.VectorSubcoreMesh(core_axis_name="c", subcore_axis_name="s"),
    compiler_params=pltpu.CompilerParams(needs_layout_passes=False),
    scratch_types=[
        pltpu.VMEM((ROWS, D), jnp.float32),
        pltpu.VMEM((D,), jnp.float32),
        pltpu.VMEM((LANES,), jnp.float32),
        pltpu.VMEM((ROWS,), jnp.float32),
        pltpu.VMEM_SHARED((V,), jnp.float32),
        pltpu.VMEM((V,), jnp.float32),
        pltpu.VMEM((CH,), jnp.int32),
        pltpu.VMEM((CH,), jnp.int32),
        pltpu.VMEM((CH,), jnp.float32),
        pltpu.VMEM((CH,), jnp.float32),
        pltpu.SemaphoreType.DMA,
        pltpu.SemaphoreType.DMA,
        pltpu.SemaphoreType.DMA,
        pltpu.SemaphoreType.DMA,
    ],
)(_sc_body)


def kernel(x1, table, W, b):
    xt = jnp.swapaxes(x1.astype(jnp.int32), 0, 1)
    w_vec = W.reshape(-1)
    b_vec = jnp.broadcast_to(b.reshape(-1), (LANES,))
    out_flat = _sc_gather(xt, table, w_vec, b_vec)
    return jnp.transpose(out_flat.reshape(L, B, 1), (1, 0, 2))


# speedup vs baseline: 1.3361x; 1.2241x over previous
import functools

import jax
import jax.numpy as jnp
from jax import lax
from jax.experimental import pallas as pl
from jax.experimental.pallas import tpu as pltpu
from jax.experimental.pallas import tpu_sc as plsc

B, L = 16384, 200
N = B * L
V, D = 800, 128
NC, NS = 2, 16
NW = NC * NS
LANES = 16
CH = 4096
EPR = B // CH
EPR_LOG = EPR.bit_length() - 1
TRIPS = N // (NW * CH)

ROWS = 64
ROW_TILES = 13


def _sc_body(xt_hbm, table_hbm, w_hbm, b_hbm, out_hbm,
             tab_v, w_v, b_v, q_local, q_shared, q_v,
             idx0, idx1, out0, out1, si0, si1, so0, so1):
    cid = lax.axis_index("c")
    sid = lax.axis_index("s")
    w = cid * NS + sid
    base = w * TRIPS

    def in_src(t):
        q = base + t
        r = lax.shift_right_logical(q, EPR_LOG)
        e = lax.bitwise_and(q, EPR - 1)
        return xt_hbm.at[r, pl.ds(e * CH, CH)]

    def out_dst(t):
        return out_hbm.at[pl.ds((base + t) * CH, CH)]

    pltpu.async_copy(in_src(0), idx0, si0)
    pltpu.async_copy(in_src(1), idx1, si1)

    pltpu.sync_copy(w_hbm, w_v)
    pltpu.sync_copy(b_hbm, b_v)

    @pl.when(sid < ROW_TILES)
    def _compute_rows():
        row0 = jnp.minimum(sid * ROWS, V - ROWS)
        pltpu.sync_copy(table_hbm.at[pl.ds(row0, ROWS)], tab_v)
        bias = b_v[pl.ds(0, LANES)][0]
        lane = lax.broadcasted_iota(jnp.int32, (LANES,), 0)
        wd = [w_v[pl.ds((d // LANES) * LANES, LANES)][d % LANES] for d in range(D)]
        for g in range(ROWS // LANES):
            rows = lane + g * LANES
            acc = jnp.zeros((LANES,), jnp.float32)
            for d in range(D):
                col = plsc.load_gather(tab_v, [rows, jnp.full((LANES,), d, jnp.int32)])
                acc = acc + col * wd[d]
            q_local[pl.ds(g * LANES, LANES)] = jnp.maximum(acc + bias, 0.0)
        pltpu.sync_copy(q_local, q_shared.at[pl.ds(row0, ROWS)])

    plsc.subcore_barrier()
    pltpu.sync_copy(q_shared, q_v)

    def gather_chunk(idx_b, out_b):
        def _inner(m, carry):
            mb = m * (16 * LANES)
            ivs = [idx_b[pl.ds(mb + jb * LANES, LANES)] for jb in range(16)]
            gs = [plsc.load_gather(q_v, [iv]) for iv in ivs]
            for jb in range(16):
                out_b[pl.ds(mb + jb * LANES, LANES)] = gs[jb]
            return carry

        lax.fori_loop(0, CH // (16 * LANES), _inner, 0)

    def wait_in(idx_b, si_b):
        pltpu.make_async_copy(xt_hbm.at[0, pl.ds(0, CH)], idx_b, si_b).wait()

    def wait_out(out_b, so_b):
        pltpu.make_async_copy(out_b, out_hbm.at[pl.ds(0, CH)], so_b).wait()

    wait_in(idx0, si0)
    gather_chunk(idx0, out0)
    pltpu.async_copy(out0, out_dst(0), so0)
    pltpu.async_copy(in_src(2), idx0, si0)

    wait_in(idx1, si1)
    gather_chunk(idx1, out1)
    pltpu.async_copy(out1, out_dst(1), so1)
    pltpu.async_copy(in_src(3), idx1, si1)

    @pl.loop(2, TRIPS - 3, step=2)
    def _steady(t):
        wait_in(idx0, si0)
        wait_out(out0, so0)
        gather_chunk(idx0, out0)
        pltpu.async_copy(out0, out_dst(t), so0)
        pltpu.async_copy(in_src(t + 2), idx0, si0)

        wait_in(idx1, si1)
        wait_out(out1, so1)
        gather_chunk(idx1, out1)
        pltpu.async_copy(out1, out_dst(t + 1), so1)
        pltpu.async_copy(in_src(t + 3), idx1, si1)

    wait_in(idx0, si0)
    wait_out(out0, so0)
    gather_chunk(idx0, out0)
    pltpu.async_copy(out0, out_dst(TRIPS - 3), so0)
    pltpu.async_copy(in_src(TRIPS - 1), idx0, si0)

    wait_in(idx1, si1)
    wait_out(out1, so1)
    gather_chunk(idx1, out1)
    pltpu.async_copy(out1, out_dst(TRIPS - 2), so1)

    wait_in(idx0, si0)
    wait_out(out0, so0)
    gather_chunk(idx0, out0)
    pltpu.async_copy(out0, out_dst(TRIPS - 1), so0)

    wait_out(out0, so0)
    wait_out(out1, so1)


_sc_gather = functools.partial(
    pl.kernel,
    out_type=jax.ShapeDtypeStruct((N,), jnp.float32),
    mesh=plsc.VectorSubcoreMesh(core_axis_name="c", subcore_axis_name="s"),
    compiler_params=pltpu.CompilerParams(needs_layout_passes=False),
    scratch_types=[
        pltpu.VMEM((ROWS, D), jnp.float32),
        pltpu.VMEM((D,), jnp.float32),
        pltpu.VMEM((LANES,), jnp.float32),
        pltpu.VMEM((ROWS,), jnp.float32),
        pltpu.VMEM_SHARED((V,), jnp.float32),
        pltpu.VMEM((V,), jnp.float32),
        pltpu.VMEM((CH,), jnp.int32),
        pltpu.VMEM((CH,), jnp.int32),
        pltpu.VMEM((CH,), jnp.float32),
        pltpu.VMEM((CH,), jnp.float32),
        pltpu.SemaphoreType.DMA,
        pltpu.SemaphoreType.DMA,
        pltpu.SemaphoreType.DMA,
        pltpu.SemaphoreType.DMA,
    ],
)(_sc_body)


def kernel(x1, table, W, b):
    xt = jnp.swapaxes(x1.astype(jnp.int32), 0, 1)
    w_vec = W.reshape(-1)
    b_vec = jnp.broadcast_to(b.reshape(-1), (LANES,))
    out_flat = _sc_gather(xt, table, w_vec, b_vec)
    return jnp.transpose(out_flat.reshape(L, B, 1), (1, 0, 2))
